# baseline (speedup 1.0000x reference)
import functools

import jax
import jax.numpy as jnp
from jax import lax
from jax.experimental import pallas as pl
from jax.experimental.pallas import tpu as pltpu
from jax.experimental.pallas import tpu_sc as plsc

B = 9216
D = 256
NKEYS = 1024
BLK = 512

STRIPES = (2048, 4096, 3072)
OFFSETS = (0, 2048, 6144)

NC, NS = 2, 16
NW = NC * NS


def _argmin_body(f_ref, k_ref, fn_ref, kn_ref, out_ref):
    f = f_ref[...]
    kw = k_ref[...]
    mm = lax.dot_general(f, kw, (((1,), (1,)), ((), ())),
                         preferred_element_type=jnp.float32)
    d = (fn_ref[...] + kn_ref[...]) - 2.0 * mm
    dmin = jnp.min(d, axis=1, keepdims=True)
    ii = lax.broadcasted_iota(jnp.int32, d.shape, 1)
    out_ref[...] = jnp.min(jnp.where(d == dmin, ii, NKEYS), axis=1)


def _argmin_tc(flat, key_weights, fnorm, knorm, stripe):
    nblk = STRIPES[stripe] // BLK
    blk_off = OFFSETS[stripe] // BLK
    idx = pl.pallas_call(
        _argmin_body,
        grid=(nblk,),
        in_specs=[
            pl.BlockSpec((BLK, D), lambda i: (i + blk_off, 0)),
            pl.BlockSpec((NKEYS, D), lambda i: (0, 0)),
            pl.BlockSpec((BLK, 1), lambda i: (i + blk_off, 0)),
            pl.BlockSpec((1, NKEYS), lambda i: (0, 0)),
        ],
        out_specs=pl.BlockSpec((BLK,), lambda i: (i,)),
        out_shape=jax.ShapeDtypeStruct((STRIPES[stripe],), jnp.int32),
    )(flat, key_weights, fnorm, knorm)
    return idx


@functools.cache
def _make_gather_add_sc(stripe):
    b_s = STRIPES[stripe]
    chunk = b_s // NW

    @functools.partial(
        pl.kernel,
        mesh=plsc.VectorSubcoreMesh(core_axis_name="c", subcore_axis_name="s"),
        scratch_types=[
            pltpu.VMEM((chunk,), jnp.int32),
            pltpu.VMEM((chunk, D), jnp.float32),
            pltpu.VMEM((chunk, D), jnp.float32),
            pltpu.SemaphoreType.DMA,
            pltpu.SemaphoreType.DMA,
        ],
    )
    def _gather_add_sc(flat_hbm, idx_hbm, val_hbm, out_hbm, idx_v, rows_v,
                       flat_v, gsem, fsem):
        wid = lax.axis_index("s") * NC + lax.axis_index("c")
        base = wid * chunk
        fcopy = pltpu.async_copy(
            flat_hbm.at[pl.ds(OFFSETS[stripe] + base, chunk)], flat_v, fsem)
        pltpu.sync_copy(idx_hbm.at[pl.ds(base, chunk)], idx_v)
        gather = pltpu.async_copy(val_hbm.at[idx_v], rows_v, gsem)
        fcopy.wait()
        gather.wait()

        @plsc.parallel_loop(0, chunk, 1, unroll=4)
        def _add(r):
            for j in range(D // 16):
                sl = pl.ds(j * 16, 16)
                rows_v[r, sl] = rows_v[r, sl] + flat_v[r, sl]

        pltpu.sync_copy(
            rows_v, out_hbm.at[pl.ds(OFFSETS[stripe] + base, chunk)])

    return _gather_add_sc


def kernel(inputs, key_weights, value_weights):
    size = inputs.shape
    flat = inputs.reshape(-1, D)
    fnorm = jnp.sum(flat ** 2, axis=-1, keepdims=True)
    knorm = jnp.sum(key_weights ** 2, axis=-1)[None, :]
    out_ref = jax.new_ref(lax.empty((B, D), jnp.float32))
    for r in range(len(STRIPES)):
        idx = _argmin_tc(flat, key_weights, fnorm, knorm, r)
        _make_gather_add_sc(r)(flat, idx, value_weights, out_ref)
    return out_ref[...].reshape(size)
